# baseline (speedup 1.0000x reference)
import functools

import jax
import jax.numpy as jnp
from jax import lax
from jax.experimental import pallas as pl
from jax.experimental.pallas import tpu as pltpu
from jax.experimental.pallas import tpu_sc as plsc

NC = 2
NS = 16
LANES = 16


def _make_kernel(N, V, D, P, T, B, n_per_w, C, G):
    n_chunks = n_per_w // C
    BW = n_per_w // T
    W2 = 2 * D
    CH = ((C + LANES - 1) // LANES) * LANES
    NG = C // LANES
    mesh = plsc.VectorSubcoreMesh(
        core_axis_name="c", subcore_axis_name="s", num_cores=NC, num_subcores=NS
    )

    @functools.partial(
        pl.kernel,
        mesh=mesh,
        compiler_params=pltpu.CompilerParams(
            use_tc_tiling_on_sc=False, needs_layout_passes=False),
        out_type=jax.ShapeDtypeStruct((N, D), jnp.float32),
        scratch_types=[
            pltpu.VMEM((C, D), jnp.float32),
            pltpu.VMEM((T, BW // 2), jnp.int32),
            pltpu.VMEM((n_per_w + LANES,), jnp.int32),
            pltpu.VMEM((CH,), jnp.int32),
            pltpu.VMEM((CH,), jnp.int32),
            pltpu.VMEM((C, W2), jnp.float32),
            pltpu.VMEM((C, W2), jnp.float32),
            pltpu.VMEM((C, D), jnp.float32),
            pltpu.VMEM((C, D), jnp.float32),
            pltpu.SemaphoreType.DMA,
            pltpu.SemaphoreType.DMA,
            pltpu.SemaphoreType.DMA,
            pltpu.SemaphoreType.DMA,
        ],
    )
    def ker(ids_hbm, tab_hbm, pos_hbm, out_hbm, pos_v, idst_v, idx_v,
            ih0, ih1, rows0, rows1, ob0, ob1, gsem0, gsem1, osem0, osem1):
        rows = (rows0, rows1)
        obuf = (ob0, ob1)
        idxh = (ih0, ih1)
        gsem = (gsem0, gsem1)
        osem = (osem0, osem1)
        wid = lax.axis_index("s") * NC + lax.axis_index("c")
        base = wid * n_per_w
        pltpu.sync_copy(pos_hbm.at[pl.ds(0, T)], pos_v)

        iota16 = lax.iota(jnp.int32, LANES)
        idx_v[pl.ds(n_per_w, LANES)] = iota16 * 0

        tcols = [(lax.rem(iota16 + (k * LANES), T),
                  lax.div(iota16 + (k * LANES), T))
                 for k in range(2 * T // LANES)]
        for h in range(2):
            pltpu.sync_copy(
                ids_hbm.at[:, pl.ds(wid * BW + h * (BW // 2), BW // 2)],
                idst_v)

            @plsc.parallel_loop(0, BW // 4, 1)
            def trans_body(bq):
                bp = bq + h * (BW // 4)
                for k, (tv, cd) in enumerate(tcols):
                    vals = plsc.load_gather(idst_v, [tv, cd + bq * 2])
                    idx_v[pl.ds(bp * (2 * T) + k * LANES, LANES)] = vals

        def fire_gathers(g, b):
            for k in range(CH // LANES):
                sl = pl.ds(k * LANES, LANES)
                idxh[b][sl] = lax.shift_right_logical(
                    idx_v[pl.ds(g * C + k * LANES, LANES)], 1)
            for j in range(C // G):
                pltpu.async_copy(
                    tab_hbm.at[idxh[b].at[pl.ds(j * G, G)]],
                    rows[b].at[pl.ds(j * G, G)],
                    gsem[b],
                )

        def drain_g(b):
            pltpu.make_async_copy(
                tab_hbm.at[pl.ds(0, C)], rows[b], gsem[b]).wait()

        def drain_o(b):
            pltpu.make_async_copy(
                out_hbm.at[pl.ds(0, C)], obuf[b], osem[b]).wait()

        def compute(g, b):
            def grp(r0, nrow):
                par = lax.mul(
                    lax.rem(idx_v[pl.ds(base * 0 + g * C + r0, LANES)], 2), D)
                for rr in range(nrow):
                    p = par[rr]
                    r = r0 + rr
                    for j in range(D // LANES):
                        sl = pl.ds(j * LANES, LANES)
                        obuf[b][r, sl] = (rows[b][r, pl.ds(p + j * LANES,
                                                           LANES)]
                                          + pos_v[r, sl])

            @plsc.parallel_loop(0, NG, 1)
            def grp_body(gr):
                grp(gr * LANES, LANES)

            if C % LANES:
                grp(NG * LANES, C % LANES)

        fire_gathers(0, 0)

        def pair_body(gp, carry):
            for b in (0, 1):
                g = gp * 2 + b
                o = 1 - b
                drain_g(b)

                @pl.when(g + 1 < n_chunks)
                def _():
                    fire_gathers(g + 1, o)

                @pl.when(g >= 2)
                def _():
                    drain_o(b)

                compute(g, b)
                pltpu.async_copy(
                    obuf[b], out_hbm.at[pl.ds(base + g * C, C)], osem[b]
                )
            return carry

        lax.fori_loop(0, n_chunks // 2, pair_body, 0)
        drain_o(0)
        drain_o(1)

    return ker


def kernel(input_ids, semantic_table, pos_table):
    B, T = input_ids.shape
    V, D = semantic_table.shape
    P = pos_table.shape[0]
    N = B * T
    NW = NC * NS
    n_per_w = N // NW
    C = T
    G = 40
    assert N % NW == 0 and n_per_w % C == 0 and C % G == 0 and D % LANES == 0
    assert (n_per_w // C) % 2 == 0 and G % 8 == 0 and V % 2 == 0
    assert n_per_w % T == 0 and (2 * T) % LANES == 0 and (n_per_w // T) % 4 == 0

    ker = _make_kernel(N, V, D, P, T, B, n_per_w, C, G)
    out_flat = ker(
        jnp.swapaxes(input_ids, 0, 1),
        semantic_table.reshape(V // 2, 2 * D),
        pos_table,
    )
    return out_flat.reshape(B, T, D)


# speedup vs baseline: 1.1501x; 1.1501x over previous
import functools

import jax
import jax.numpy as jnp
from jax import lax
from jax.experimental import pallas as pl
from jax.experimental.pallas import tpu as pltpu
from jax.experimental.pallas import tpu_sc as plsc

NC = 2
NS = 16
LANES = 16


def _make_kernel(N, V, D, P, T, B, n_per_w, C, G):
    n_chunks = n_per_w // C
    BW = n_per_w // T
    W2 = 2 * D
    CH = ((C + LANES - 1) // LANES) * LANES
    NG = C // LANES
    mesh = plsc.VectorSubcoreMesh(
        core_axis_name="c", subcore_axis_name="s", num_cores=NC, num_subcores=NS
    )

    @functools.partial(
        pl.kernel,
        mesh=mesh,
        compiler_params=pltpu.CompilerParams(
            use_tc_tiling_on_sc=True, needs_layout_passes=False),
        out_type=jax.ShapeDtypeStruct((N, D), jnp.float32),
        scratch_types=[
            pltpu.VMEM((((T // 2 + 7) // 8) * 8, W2), jnp.float32),
            pltpu.VMEM((T, BW), jnp.int32),
            pltpu.VMEM((n_per_w + LANES,), jnp.int32),
            pltpu.VMEM((CH,), jnp.int32),
            pltpu.VMEM((CH,), jnp.int32),
            pltpu.VMEM((C, W2), jnp.float32),
            pltpu.VMEM((C, W2), jnp.float32),
            pltpu.VMEM((C, D), jnp.float32),
            pltpu.VMEM((C, D), jnp.float32),
            pltpu.SemaphoreType.DMA,
            pltpu.SemaphoreType.DMA,
            pltpu.SemaphoreType.DMA,
            pltpu.SemaphoreType.DMA,
        ],
    )
    def ker(ids_hbm, tab_hbm, pos_hbm, out_hbm, pos_v, idst_v, idx_v,
            ih0, ih1, rows0, rows1, ob0, ob1, gsem0, gsem1, osem0, osem1):
        rows = (rows0, rows1)
        obuf = (ob0, ob1)
        idxh = (ih0, ih1)
        gsem = (gsem0, gsem1)
        osem = (osem0, osem1)
        wid = lax.axis_index("s") * NC + lax.axis_index("c")
        base = wid * n_per_w
        ph = ((T // 2 + 7) // 8) * 8
        pltpu.sync_copy(pos_hbm.at[pl.ds(0, ph)], pos_v)

        iota16 = lax.iota(jnp.int32, LANES)
        idx_v[pl.ds(n_per_w, LANES)] = iota16 * 0

        tcols = [(lax.rem(iota16 + (k * LANES), T),
                  lax.div(iota16 + (k * LANES), T))
                 for k in range(2 * T // LANES)]
        pltpu.sync_copy(ids_hbm.at[:, pl.ds(wid * BW, BW)], idst_v)

        @plsc.parallel_loop(0, BW // 2, 1)
        def trans_body(bp):
            for k, (tv, cd) in enumerate(tcols):
                vals = plsc.load_gather(idst_v, [tv, cd + bp * 2])
                idx_v[pl.ds(bp * (2 * T) + k * LANES, LANES)] = vals

        def fire_gathers(g, b):
            for k in range(CH // LANES):
                sl = pl.ds(k * LANES, LANES)
                idxh[b][sl] = lax.shift_right_logical(
                    idx_v[pl.ds(g * C + k * LANES, LANES)], 1)
            for j in range(C // G):
                pltpu.async_copy(
                    tab_hbm.at[idxh[b].at[pl.ds(j * G, G)]],
                    rows[b].at[pl.ds(j * G, G)],
                    gsem[b],
                )

        def drain_g(b):
            pltpu.make_async_copy(
                tab_hbm.at[pl.ds(0, C)], rows[b], gsem[b]).wait()

        def drain_o(b):
            pltpu.make_async_copy(
                out_hbm.at[pl.ds(0, C)], obuf[b], osem[b]).wait()

        def compute(g, b):
            @plsc.parallel_loop(0, NG, 1)
            def grp_body(gr):
                r0 = gr * LANES
                par = lax.mul(
                    lax.rem(idx_v[pl.ds(g * C + r0, LANES)], 2), D)
                for rr in range(LANES):
                    p = par[rr]
                    r = r0 + rr
                    t = lax.rem(g * C + r, T)
                    th = lax.shift_right_logical(t, 1)
                    toff = lax.mul(lax.rem(t, 2), D)
                    for j in range(D // LANES):
                        obuf[b][r, pl.ds(j * LANES, LANES)] = (
                            rows[b][r, pl.ds(p + j * LANES, LANES)]
                            + pos_v[th, pl.ds(toff + j * LANES, LANES)])

        fire_gathers(0, 0)

        def pair_body(gp, carry):
            for b in (0, 1):
                g = gp * 2 + b
                o = 1 - b
                drain_g(b)

                @pl.when(g + 1 < n_chunks)
                def _():
                    fire_gathers(g + 1, o)

                @pl.when(g >= 2)
                def _():
                    drain_o(b)

                compute(g, b)
                pltpu.async_copy(
                    obuf[b], out_hbm.at[pl.ds(base + g * C, C)], osem[b]
                )
            return carry

        lax.fori_loop(0, n_chunks // 2, pair_body, 0)
        drain_o(0)
        drain_o(1)

    return ker


def kernel(input_ids, semantic_table, pos_table):
    B, T = input_ids.shape
    V, D = semantic_table.shape
    P = pos_table.shape[0]
    N = B * T
    NW = NC * NS
    n_per_w = N // NW
    C = 128
    G = 128
    assert N % NW == 0 and n_per_w % C == 0 and C % G == 0 and D % LANES == 0
    assert (n_per_w // C) % 2 == 0 and G % 8 == 0 and V % 2 == 0
    assert n_per_w % T == 0 and (2 * T) % LANES == 0 and (n_per_w // T) % 4 == 0

    ker = _make_kernel(N, V, D, P, T, B, n_per_w, C, G)
    out_flat = ker(
        jnp.swapaxes(input_ids, 0, 1),
        semantic_table.reshape(V // 2, 2 * D),
        pos_table.reshape(P // 2, 2 * D),
    )
    return out_flat.reshape(B, T, D)
